# Initial kernel scaffold; baseline (speedup 1.0000x reference)
import jax
import jax.numpy as jnp
from jax.experimental import pallas as pl


def kernel(x, edge_index, edge_weight, W, b):
    raise NotImplementedError("write your pallas kernel here")


# speedup vs baseline: 3.4397x; 3.4397x over previous
import functools

import jax
import jax.numpy as jnp
from jax import lax
from jax.experimental import pallas as pl
from jax.experimental.pallas import tpu as pltpu
from jax.experimental.pallas import tpu_sc as plsc

N_NODES = 10000
N_PAD = 10240
D_IN = 256
D_OUT = 256
HALF = 128
NC, NS = 2, 16
CHUNK = 128
GRP = 16
LANES = 16

_BCAST_DNUMS = lax.GatherDimensionNumbers(
    offset_dims=(), collapsed_slice_dims=(0,), start_index_map=(0,))


def _lane_broadcast(vec, lane):
    idx = jnp.full((LANES, 1), lane, jnp.int32)
    return lax.gather(vec, idx, _BCAST_DNUMS, (1,),
                      mode=lax.GatherScatterMode.PROMISE_IN_BOUNDS)


def _matmul_body(x_ref, w_ref, out_ref):
    out_ref[0] = jnp.dot(x_ref[...], w_ref[...],
                         preferred_element_type=jnp.float32)


def _tc_support(x, W):
    br = 400
    return pl.pallas_call(
        _matmul_body,
        grid=(N_NODES // br, NC),
        in_specs=[
            pl.BlockSpec((br, D_IN), lambda i, h: (i, 0)),
            pl.BlockSpec((D_IN, HALF), lambda i, h: (0, h)),
        ],
        out_specs=pl.BlockSpec((1, br, HALF), lambda i, h: (h, i, 0)),
        out_shape=jax.ShapeDtypeStruct((NC, N_NODES, HALF), jnp.float32),
    )(x, W)


def _make_sc_spmm(cpt):
    rows_per_tile = N_PAD // NS
    ngrp = cpt // GRP
    mesh = plsc.VectorSubcoreMesh(core_axis_name="c", subcore_axis_name="s",
                                  num_cores=NC, num_subcores=NS)

    idx_set = [pltpu.VMEM((GRP, CHUNK), jnp.int32),
               pltpu.VMEM((GRP, CHUNK), jnp.int32),
               pltpu.VMEM((GRP, CHUNK), jnp.float32),
               pltpu.SemaphoreType.DMA]

    @functools.partial(
        pl.kernel,
        out_type=jax.ShapeDtypeStruct((N_PAD, D_OUT), jnp.float32),
        mesh=mesh,
        scratch_types=[
            pltpu.VMEM_SHARED((N_PAD, HALF), jnp.float32),
            pltpu.VMEM((CHUNK, HALF), jnp.float32),
            pltpu.VMEM((CHUNK, HALF), jnp.float32),
            pltpu.VMEM((HALF,), jnp.float32),
            pltpu.SemaphoreType.DMA,
            pltpu.SemaphoreType.DMA,
        ] + idx_set + idx_set,
    )
    def sc_spmm(src_ref, dst_ref, ew_ref, b_ref, sup_ref, out_ref,
                acc, g0, g1, bbuf, gsem0, gsem1,
                srcA, dstA, ewA, isemA, srcB, dstB, ewB, isemB):
        c = lax.axis_index("c")
        s = lax.axis_index("s")
        row0 = s * rows_per_tile

        def start_idx(gg, sbuf, dbuf, wbuf, isem):
            base = s * cpt + gg * GRP
            pltpu.async_copy(src_ref.at[c, pl.ds(base, GRP)], sbuf, isem)
            pltpu.async_copy(dst_ref.at[pl.ds(base, GRP)], dbuf, isem)
            pltpu.async_copy(ew_ref.at[pl.ds(base, GRP)], wbuf, isem)

        def wait_idx(sbuf, dbuf, wbuf, isem):
            pltpu.make_async_copy(src_ref.at[0, pl.ds(0, GRP)], sbuf,
                                  isem).wait()
            pltpu.make_async_copy(dst_ref.at[pl.ds(0, GRP)], dbuf,
                                  isem).wait()
            pltpu.make_async_copy(ew_ref.at[pl.ds(0, GRP)], wbuf,
                                  isem).wait()

        pltpu.sync_copy(b_ref.at[pl.ds(c * HALF, HALF)], bbuf)
        bv = [bbuf[pl.ds(c8 * LANES, LANES)] for c8 in range(HALF // LANES)]

        @pl.loop(0, CHUNK)
        def _fill(r):
            for c8 in range(HALF // LANES):
                g0[r, pl.ds(c8 * LANES, LANES)] = bv[c8]

        for k in range(rows_per_tile // CHUNK):
            pltpu.sync_copy(g0, acc.at[pl.ds(row0 + k * CHUNK, CHUNK)])
        plsc.subcore_barrier()

        start_idx(0, srcA, dstA, ewA, isemA)
        start_idx(1, srcB, dstB, ewB, isemB)

        def process(kk, sbuf, dbuf, wbuf, buf, gsem):
            pltpu.make_async_copy(sup_ref.at[sbuf.at[0]], buf, gsem).wait()

            @pl.loop(0, CHUNK // LANES)
            def _scale(g):
                wgrp = wbuf[kk, pl.ds(g * LANES, LANES)]
                for u in range(LANES):
                    e = g * LANES + u
                    wv = _lane_broadcast(wgrp, u)
                    for c8 in range(HALF // LANES):
                        sl = pl.ds(c8 * LANES, LANES)
                        buf[e, sl] = buf[e, sl] * wv

            pltpu.sync_copy(buf, acc.at[dbuf.at[kk]], add=True)

            @pl.when(kk < GRP - 2)
            def _():
                pltpu.async_copy(sup_ref.at[sbuf.at[kk + 2]], buf, gsem)

        def do_group(gg, sbuf, dbuf, wbuf, isem):
            wait_idx(sbuf, dbuf, wbuf, isem)
            pltpu.async_copy(sup_ref.at[sbuf.at[0]], g0, gsem0)
            pltpu.async_copy(sup_ref.at[sbuf.at[1]], g1, gsem1)

            @pl.loop(0, GRP, step=2)
            def _chunks(k):
                process(k, sbuf, dbuf, wbuf, g0, gsem0)
                process(k + 1, sbuf, dbuf, wbuf, g1, gsem1)

            @pl.when(gg + 2 < ngrp)
            def _():
                start_idx(gg + 2, sbuf, dbuf, wbuf, isem)

        @pl.loop(0, ngrp - (ngrp % 2), step=2)
        def _groups(gg):
            do_group(gg, srcA, dstA, ewA, isemA)
            do_group(gg + 1, srcB, dstB, ewB, isemB)

        if ngrp % 2:
            do_group(ngrp - 1, srcA, dstA, ewA, isemA)

        plsc.subcore_barrier()
        pltpu.sync_copy(
            acc.at[pl.ds(row0, rows_per_tile)],
            out_ref.at[pl.ds(row0, rows_per_tile), pl.ds(c * HALF, HALF)])

    return sc_spmm


def kernel(x, edge_index, edge_weight, W, b):
    support = _tc_support(x, W).reshape(NC * N_NODES, HALF)

    e = edge_index.shape[1]
    per_tile = -(-e // (NS * CHUNK))
    cpt = -(-per_tile // GRP) * GRP
    e_pad = NS * cpt * CHUNK
    ei = jnp.pad(edge_index, ((0, 0), (0, e_pad - e)))
    ew = jnp.pad(edge_weight, (0, e_pad - e)).reshape(NS * cpt, CHUNK)
    src = ei[0].reshape(NS * cpt, CHUNK)
    srcs = jnp.stack([src, src + N_NODES])
    dst = ei[1].reshape(NS * cpt, CHUNK)

    return _make_sc_spmm(cpt)(srcs, dst, ew, b, support)[:N_NODES]


# speedup vs baseline: 5.3898x; 1.5670x over previous
import functools

import jax
import jax.numpy as jnp
from jax import lax
from jax.experimental import pallas as pl
from jax.experimental.pallas import tpu as pltpu
from jax.experimental.pallas import tpu_sc as plsc

N_NODES = 10000
D_IN = 256
D_OUT = 256
HALF = 128
NC, NS = 2, 16
CHUNK = 128
RING = 4
LANES = 16
ROWS_A = 624
ROWS_B = 640

_BCAST_DNUMS = lax.GatherDimensionNumbers(
    offset_dims=(), collapsed_slice_dims=(0,), start_index_map=(0,))


def _lane_broadcast(vec, lane):
    idx = jnp.full((LANES, 1), lane, jnp.int32)
    return lax.gather(vec, idx, _BCAST_DNUMS, (1,),
                      mode=lax.GatherScatterMode.PROMISE_IN_BOUNDS)


def _matmul_body(x_ref, w_ref, out_ref):
    out_ref[0] = jnp.dot(x_ref[...], w_ref[...],
                         preferred_element_type=jnp.float32)


def _tc_support(x, W):
    br = 400
    return pl.pallas_call(
        _matmul_body,
        grid=(N_NODES // br, NC),
        in_specs=[
            pl.BlockSpec((br, D_IN), lambda i, h: (i, 0)),
            pl.BlockSpec((D_IN, HALF), lambda i, h: (0, h)),
        ],
        out_specs=pl.BlockSpec((1, br, HALF), lambda i, h: (h, i, 0)),
        out_shape=jax.ShapeDtypeStruct((NC, N_NODES, HALF), jnp.float32),
    )(x, W)


def _make_sc_spmm(cpt):
    mesh = plsc.VectorSubcoreMesh(core_axis_name="c", subcore_axis_name="s",
                                  num_cores=NC, num_subcores=NS)

    @functools.partial(
        pl.kernel,
        out_type=jax.ShapeDtypeStruct((N_NODES, D_OUT), jnp.float32),
        mesh=mesh,
        scratch_types=[
            pltpu.VMEM_SHARED((N_NODES, HALF), jnp.float32),
            pltpu.VMEM((CHUNK, HALF), jnp.float32),
            pltpu.VMEM((CHUNK, HALF), jnp.float32),
            pltpu.VMEM((CHUNK, HALF), jnp.float32),
            pltpu.VMEM((RING, CHUNK), jnp.int32),
            pltpu.VMEM((RING, CHUNK), jnp.int32),
            pltpu.VMEM((RING, CHUNK), jnp.float32),
            pltpu.VMEM((HALF,), jnp.float32),
            pltpu.SemaphoreType.DMA,
            pltpu.SemaphoreType.DMA,
            pltpu.SemaphoreType.DMA,
            pltpu.SemaphoreType.DMA,
            pltpu.SemaphoreType.DMA,
        ],
    )
    def sc_spmm(src_ref, dst_ref, ew_ref, b_ref, sup_ref, out_ref,
                acc, b0, b1, b2, src_g, dst_g, w_g, bbuf,
                gsem, ssem, isem_s, isem_d, isem_w):
        c = lax.axis_index("c")
        s = lax.axis_index("s")
        bufs = (b0, b1, b2)
        row0 = s * ROWS_A

        def fire_idx(k):
            slot = k & 3
            base = s * cpt + k
            pltpu.async_copy(src_ref.at[c, base], src_g.at[slot], isem_s)
            pltpu.async_copy(dst_ref.at[base], dst_g.at[slot], isem_d)
            pltpu.async_copy(ew_ref.at[base], w_g.at[slot], isem_w)

        def wait_idx():
            pltpu.make_async_copy(src_ref.at[0, 0], src_g.at[0],
                                  isem_s).wait()
            pltpu.make_async_copy(dst_ref.at[0], dst_g.at[0], isem_d).wait()
            pltpu.make_async_copy(ew_ref.at[0], w_g.at[0], isem_w).wait()

        def wait_gather(buf):
            pltpu.make_async_copy(sup_ref.at[src_g.at[0]], buf, gsem).wait()

        def wait_scatter():
            pltpu.make_async_copy(b0, acc.at[dst_g.at[0]], ssem).wait()

        pltpu.sync_copy(b_ref.at[pl.ds(c * HALF, HALF)], bbuf)
        bv = [bbuf[pl.ds(c8 * LANES, LANES)] for c8 in range(HALF // LANES)]

        @pl.loop(0, CHUNK)
        def _fill(r):
            for c8 in range(HALF // LANES):
                b0[r, pl.ds(c8 * LANES, LANES)] = bv[c8]

        @pl.when(s < NS - 1)
        def _():
            for k in range(ROWS_A // CHUNK):
                pltpu.sync_copy(b0, acc.at[pl.ds(row0 + k * CHUNK, CHUNK)])
            rem = ROWS_A % CHUNK
            pltpu.sync_copy(
                b0.at[pl.ds(0, rem)],
                acc.at[pl.ds(row0 + (ROWS_A // CHUNK) * CHUNK, rem)])

        @pl.when(s == NS - 1)
        def _():
            for k in range(ROWS_B // CHUNK):
                pltpu.sync_copy(
                    b0, acc.at[pl.ds((NS - 1) * ROWS_A + k * CHUNK, CHUNK)])

        plsc.subcore_barrier()

        fire_idx(0)
        fire_idx(1)
        fire_idx(2)
        wait_idx()
        pltpu.async_copy(sup_ref.at[src_g.at[0]], b0, gsem)
        wait_idx()
        pltpu.async_copy(sup_ref.at[src_g.at[1]], b1, gsem)

        @pl.loop(0, cpt)
        def _step(k):
            @pl.when(k >= 1)
            def _():
                wait_scatter()

            @pl.when(k + 3 < cpt)
            def _():
                fire_idx(k + 3)

            for i in range(3):
                @pl.when(lax.rem(k, 3) == i)
                def _(i=i):
                    buf = bufs[i]
                    nbuf = bufs[(i + 2) % 3]

                    @pl.when(k + 2 < cpt)
                    def _():
                        wait_idx()
                        pltpu.async_copy(sup_ref.at[src_g.at[(k + 2) & 3]],
                                         nbuf, gsem)

                    wait_gather(buf)
                    slot = k & 3

                    @pl.loop(0, CHUNK // LANES)
                    def _scale(g):
                        wgrp = w_g[slot, pl.ds(g * LANES, LANES)]
                        for u in range(LANES):
                            e = g * LANES + u
                            wv = _lane_broadcast(wgrp, u)
                            for c8 in range(HALF // LANES):
                                sl = pl.ds(c8 * LANES, LANES)
                                buf[e, sl] = buf[e, sl] * wv

                    pltpu.async_copy(buf, acc.at[dst_g.at[slot]], ssem,
                                     add=True)

        wait_scatter()
        plsc.subcore_barrier()

        @pl.when(s < NS - 1)
        def _():
            pltpu.sync_copy(
                acc.at[pl.ds(row0, ROWS_A)],
                out_ref.at[pl.ds(row0, ROWS_A), pl.ds(c * HALF, HALF)])

        @pl.when(s == NS - 1)
        def _():
            pltpu.sync_copy(
                acc.at[pl.ds((NS - 1) * ROWS_A, ROWS_B)],
                out_ref.at[pl.ds((NS - 1) * ROWS_A, ROWS_B),
                           pl.ds(c * HALF, HALF)])

    return sc_spmm


def kernel(x, edge_index, edge_weight, W, b):
    support = _tc_support(x, W).reshape(NC * N_NODES, HALF)

    e = edge_index.shape[1]
    cpt = -(-e // (NS * CHUNK))
    e_pad = NS * cpt * CHUNK
    ei = jnp.pad(edge_index, ((0, 0), (0, e_pad - e)))
    ew = jnp.pad(edge_weight, (0, e_pad - e)).reshape(NS * cpt, CHUNK)
    src = ei[0].reshape(NS * cpt, CHUNK)
    srcs = jnp.stack([src, src + N_NODES])
    dst = ei[1].reshape(NS * cpt, CHUNK)

    return _make_sc_spmm(cpt)(srcs, dst, ew, b, support)


# speedup vs baseline: 5.4837x; 1.0174x over previous
import functools

import jax
import jax.numpy as jnp
from jax import lax
from jax.experimental import pallas as pl
from jax.experimental.pallas import tpu as pltpu
from jax.experimental.pallas import tpu_sc as plsc

N_NODES = 10000
D_IN = 256
D_OUT = 256
HALF = 128
NC, NS = 2, 16
CHUNK = 128
RING = 4
LANES = 16
ROWS_A = 624
ROWS_B = 640

_BCAST_DNUMS = lax.GatherDimensionNumbers(
    offset_dims=(), collapsed_slice_dims=(0,), start_index_map=(0,))


def _lane_broadcast(vec, lane):
    idx = jnp.full((LANES, 1), lane, jnp.int32)
    return lax.gather(vec, idx, _BCAST_DNUMS, (1,),
                      mode=lax.GatherScatterMode.PROMISE_IN_BOUNDS)


def _matmul_body(x_ref, w_ref, out_ref):
    out_ref[0] = jnp.dot(x_ref[...], w_ref[...],
                         preferred_element_type=jnp.float32)


def _tc_support(x, W):
    br = 400
    return pl.pallas_call(
        _matmul_body,
        grid=(N_NODES // br, NC),
        in_specs=[
            pl.BlockSpec((br, D_IN), lambda i, h: (i, 0)),
            pl.BlockSpec((D_IN, HALF), lambda i, h: (0, h)),
        ],
        out_specs=pl.BlockSpec((1, br, HALF), lambda i, h: (h, i, 0)),
        out_shape=jax.ShapeDtypeStruct((NC, N_NODES, HALF), jnp.float32),
    )(x, W)


def _make_sc_spmm(cpt):
    mesh = plsc.VectorSubcoreMesh(core_axis_name="c", subcore_axis_name="s",
                                  num_cores=NC, num_subcores=NS)

    @functools.partial(
        pl.kernel,
        out_type=jax.ShapeDtypeStruct((N_NODES, D_OUT), jnp.float32),
        mesh=mesh,
        scratch_types=[
            pltpu.VMEM_SHARED((N_NODES, HALF), jnp.float32),
            pltpu.VMEM((CHUNK, HALF), jnp.float32),
            pltpu.VMEM((CHUNK, HALF), jnp.float32),
            pltpu.VMEM((CHUNK, HALF), jnp.float32),
            pltpu.VMEM((RING, CHUNK), jnp.int32),
            pltpu.VMEM((RING, CHUNK), jnp.int32),
            pltpu.VMEM((RING, CHUNK), jnp.float32),
            pltpu.VMEM((HALF,), jnp.float32),
            pltpu.SemaphoreType.DMA,
            pltpu.SemaphoreType.DMA,
            pltpu.SemaphoreType.DMA,
            pltpu.SemaphoreType.DMA,
            pltpu.SemaphoreType.DMA,
        ],
    )
    def sc_spmm(src_ref, dst_ref, ew_ref, b_ref, sup_ref, out_ref,
                acc, b0, b1, b2, src_g, dst_g, w_g, bbuf,
                gsem, ssem, isem_s, isem_d, isem_w):
        c = lax.axis_index("c")
        s = lax.axis_index("s")
        bufs = (b0, b1, b2)
        row0 = s * ROWS_A

        def fire_idx(k):
            slot = k & 3
            base = s * cpt + k
            pltpu.async_copy(src_ref.at[c, base], src_g.at[slot], isem_s)
            pltpu.async_copy(dst_ref.at[base], dst_g.at[slot], isem_d)
            pltpu.async_copy(ew_ref.at[base], w_g.at[slot], isem_w)

        def wait_idx():
            pltpu.make_async_copy(src_ref.at[0, 0], src_g.at[0],
                                  isem_s).wait()
            pltpu.make_async_copy(dst_ref.at[0], dst_g.at[0], isem_d).wait()
            pltpu.make_async_copy(ew_ref.at[0], w_g.at[0], isem_w).wait()

        def wait_gather(buf):
            pltpu.make_async_copy(sup_ref.at[src_g.at[0]], buf, gsem).wait()

        def wait_scatter():
            pltpu.make_async_copy(b0, acc.at[dst_g.at[0]], ssem).wait()

        pltpu.sync_copy(b_ref.at[pl.ds(c * HALF, HALF)], bbuf)
        bv = [bbuf[pl.ds(c8 * LANES, LANES)] for c8 in range(HALF // LANES)]

        @pl.loop(0, CHUNK)
        def _fill(r):
            for c8 in range(HALF // LANES):
                b0[r, pl.ds(c8 * LANES, LANES)] = bv[c8]

        @pl.when(s < NS - 1)
        def _():
            for k in range(ROWS_A // CHUNK):
                pltpu.sync_copy(b0, acc.at[pl.ds(row0 + k * CHUNK, CHUNK)])
            rem = ROWS_A % CHUNK
            pltpu.sync_copy(
                b0.at[pl.ds(0, rem)],
                acc.at[pl.ds(row0 + (ROWS_A // CHUNK) * CHUNK, rem)])

        @pl.when(s == NS - 1)
        def _():
            for k in range(ROWS_B // CHUNK):
                pltpu.sync_copy(
                    b0, acc.at[pl.ds((NS - 1) * ROWS_A + k * CHUNK, CHUNK)])

        plsc.subcore_barrier()

        fire_idx(0)
        fire_idx(1)
        fire_idx(2)
        wait_idx()
        pltpu.async_copy(sup_ref.at[src_g.at[0]], b0, gsem)
        wait_idx()
        pltpu.async_copy(sup_ref.at[src_g.at[1]], b1, gsem)

        @pl.loop(0, cpt)
        def _step(k):
            for i in range(3):
                @pl.when(lax.rem(k, 3) == i)
                def _(i=i):
                    buf = bufs[i]
                    nbuf = bufs[(i + 2) % 3]

                    wait_gather(buf)
                    slot = k & 3

                    @pl.loop(0, CHUNK // LANES)
                    def _scale(g):
                        wgrp = w_g[slot, pl.ds(g * LANES, LANES)]
                        for u in range(LANES):
                            e = g * LANES + u
                            wv = _lane_broadcast(wgrp, u)
                            for c8 in range(HALF // LANES):
                                sl = pl.ds(c8 * LANES, LANES)
                                buf[e, sl] = buf[e, sl] * wv

                    @pl.when(k >= 1)
                    def _():
                        wait_scatter()

                    @pl.when(k + 3 < cpt)
                    def _():
                        fire_idx(k + 3)

                    @pl.when(k + 2 < cpt)
                    def _():
                        wait_idx()
                        pltpu.async_copy(sup_ref.at[src_g.at[(k + 2) & 3]],
                                         nbuf, gsem)

                    pltpu.async_copy(buf, acc.at[dst_g.at[slot]], ssem,
                                     add=True)

        wait_scatter()
        plsc.subcore_barrier()

        @pl.when(s < NS - 1)
        def _():
            pltpu.sync_copy(
                acc.at[pl.ds(row0, ROWS_A)],
                out_ref.at[pl.ds(row0, ROWS_A), pl.ds(c * HALF, HALF)])

        @pl.when(s == NS - 1)
        def _():
            pltpu.sync_copy(
                acc.at[pl.ds((NS - 1) * ROWS_A, ROWS_B)],
                out_ref.at[pl.ds((NS - 1) * ROWS_A, ROWS_B),
                           pl.ds(c * HALF, HALF)])

    return sc_spmm


def kernel(x, edge_index, edge_weight, W, b):
    support = _tc_support(x, W).reshape(NC * N_NODES, HALF)

    e = edge_index.shape[1]
    cpt = -(-e // (NS * CHUNK))
    e_pad = NS * cpt * CHUNK
    ei = jnp.pad(edge_index, ((0, 0), (0, e_pad - e)))
    ew = jnp.pad(edge_weight, (0, e_pad - e)).reshape(NS * cpt, CHUNK)
    src = ei[0].reshape(NS * cpt, CHUNK)
    srcs = jnp.stack([src, src + N_NODES])
    dst = ei[1].reshape(NS * cpt, CHUNK)

    return _make_sc_spmm(cpt)(srcs, dst, ew, b, support)


# speedup vs baseline: 5.4865x; 1.0005x over previous
import functools

import jax
import jax.numpy as jnp
from jax import lax
from jax.experimental import pallas as pl
from jax.experimental.pallas import tpu as pltpu
from jax.experimental.pallas import tpu_sc as plsc

N_NODES = 10000
D_IN = 256
D_OUT = 256
HALF = 128
NC, NS = 2, 16
CHUNK = 128
RING = 4
LANES = 16
ROWS_A = 624
ROWS_B = 640

_BCAST_DNUMS = lax.GatherDimensionNumbers(
    offset_dims=(), collapsed_slice_dims=(0,), start_index_map=(0,))


def _lane_broadcast(vec, lane):
    idx = jnp.full((LANES, 1), lane, jnp.int32)
    return lax.gather(vec, idx, _BCAST_DNUMS, (1,),
                      mode=lax.GatherScatterMode.PROMISE_IN_BOUNDS)


def _matmul_body(x_ref, w_ref, out_ref):
    out_ref[0] = jnp.dot(x_ref[...], w_ref[...],
                         preferred_element_type=jnp.float32)


def _tc_support(x, W):
    br = 400
    return pl.pallas_call(
        _matmul_body,
        grid=(N_NODES // br, NC),
        in_specs=[
            pl.BlockSpec((br, D_IN), lambda i, h: (i, 0)),
            pl.BlockSpec((D_IN, HALF), lambda i, h: (0, h)),
        ],
        out_specs=pl.BlockSpec((1, br, HALF), lambda i, h: (h, i, 0)),
        out_shape=jax.ShapeDtypeStruct((NC, N_NODES, HALF), jnp.float32),
    )(x, W)


def _make_sc_spmm(cpt):
    mesh = plsc.VectorSubcoreMesh(core_axis_name="c", subcore_axis_name="s",
                                  num_cores=NC, num_subcores=NS)

    @functools.partial(
        pl.kernel,
        out_type=jax.ShapeDtypeStruct((N_NODES, D_OUT), jnp.float32),
        mesh=mesh,
        scratch_types=[
            pltpu.VMEM_SHARED((N_NODES, HALF), jnp.float32),
            pltpu.VMEM((CHUNK, HALF), jnp.float32),
            pltpu.VMEM((CHUNK, HALF), jnp.float32),
            pltpu.VMEM((CHUNK, HALF), jnp.float32),
            pltpu.VMEM((RING, CHUNK), jnp.int32),
            pltpu.VMEM((RING, CHUNK), jnp.int32),
            pltpu.VMEM((RING, CHUNK), jnp.float32),
            pltpu.VMEM((HALF,), jnp.float32),
            pltpu.SemaphoreType.DMA,
            pltpu.SemaphoreType.DMA,
            pltpu.SemaphoreType.DMA,
            pltpu.SemaphoreType.DMA,
            pltpu.SemaphoreType.DMA,
        ],
    )
    def sc_spmm(src_ref, dst_ref, ew_ref, b_ref, sup_ref, out_ref,
                acc, b0, b1, b2, src_g, dst_g, w_g, bbuf,
                gsem, ssem, isem_s, isem_d, isem_w):
        c = lax.axis_index("c")
        s = lax.axis_index("s")
        bufs = (b0, b1, b2)
        row0 = s * ROWS_A

        def fire_idx(k):
            slot = k & 3
            base = s * cpt + k
            pltpu.async_copy(src_ref.at[c, base], src_g.at[slot], isem_s)
            pltpu.async_copy(dst_ref.at[base], dst_g.at[slot], isem_d)
            pltpu.async_copy(ew_ref.at[base], w_g.at[slot], isem_w)

        def wait_idx():
            pltpu.make_async_copy(src_ref.at[0, 0], src_g.at[0],
                                  isem_s).wait()
            pltpu.make_async_copy(dst_ref.at[0], dst_g.at[0], isem_d).wait()
            pltpu.make_async_copy(ew_ref.at[0], w_g.at[0], isem_w).wait()

        def fire_gather(slot, buf):
            pltpu.async_copy(sup_ref.at[src_g.at[slot, pl.ds(0, 64)]],
                             buf.at[pl.ds(0, 64)], gsem)
            pltpu.async_copy(sup_ref.at[src_g.at[slot, pl.ds(64, 64)]],
                             buf.at[pl.ds(64, 64)], gsem)

        def wait_gather(buf):
            pltpu.make_async_copy(sup_ref.at[src_g.at[0, pl.ds(0, 64)]],
                                  buf.at[pl.ds(0, 64)], gsem).wait()
            pltpu.make_async_copy(sup_ref.at[src_g.at[0, pl.ds(0, 64)]],
                                  buf.at[pl.ds(64, 64)], gsem).wait()

        def wait_scatter():
            pltpu.make_async_copy(b0, acc.at[dst_g.at[0]], ssem).wait()

        pltpu.sync_copy(b_ref.at[pl.ds(c * HALF, HALF)], bbuf)
        bv = [bbuf[pl.ds(c8 * LANES, LANES)] for c8 in range(HALF // LANES)]

        @pl.loop(0, CHUNK)
        def _fill(r):
            for c8 in range(HALF // LANES):
                b0[r, pl.ds(c8 * LANES, LANES)] = bv[c8]

        @pl.when(s < NS - 1)
        def _():
            for k in range(ROWS_A // CHUNK):
                pltpu.sync_copy(b0, acc.at[pl.ds(row0 + k * CHUNK, CHUNK)])
            rem = ROWS_A % CHUNK
            pltpu.sync_copy(
                b0.at[pl.ds(0, rem)],
                acc.at[pl.ds(row0 + (ROWS_A // CHUNK) * CHUNK, rem)])

        @pl.when(s == NS - 1)
        def _():
            for k in range(ROWS_B // CHUNK):
                pltpu.sync_copy(
                    b0, acc.at[pl.ds((NS - 1) * ROWS_A + k * CHUNK, CHUNK)])

        plsc.subcore_barrier()

        fire_idx(0)
        fire_idx(1)
        fire_idx(2)
        wait_idx()
        fire_gather(0, b0)
        wait_idx()
        fire_gather(1, b1)

        @pl.loop(0, cpt)
        def _step(k):
            for i in range(3):
                @pl.when(lax.rem(k, 3) == i)
                def _(i=i):
                    buf = bufs[i]
                    nbuf = bufs[(i + 2) % 3]

                    wait_gather(buf)
                    slot = k & 3

                    @pl.loop(0, CHUNK // LANES)
                    def _scale(g):
                        wgrp = w_g[slot, pl.ds(g * LANES, LANES)]
                        for u in range(LANES):
                            e = g * LANES + u
                            wv = _lane_broadcast(wgrp, u)
                            for c8 in range(HALF // LANES):
                                sl = pl.ds(c8 * LANES, LANES)
                                buf[e, sl] = buf[e, sl] * wv

                    @pl.when(k >= 1)
                    def _():
                        wait_scatter()

                    @pl.when(k + 3 < cpt)
                    def _():
                        fire_idx(k + 3)

                    @pl.when(k + 2 < cpt)
                    def _():
                        wait_idx()
                        fire_gather((k + 2) & 3, nbuf)

                    pltpu.async_copy(buf, acc.at[dst_g.at[slot]], ssem,
                                     add=True)

        wait_scatter()
        plsc.subcore_barrier()

        @pl.when(s < NS - 1)
        def _():
            pltpu.sync_copy(
                acc.at[pl.ds(row0, ROWS_A)],
                out_ref.at[pl.ds(row0, ROWS_A), pl.ds(c * HALF, HALF)])

        @pl.when(s == NS - 1)
        def _():
            pltpu.sync_copy(
                acc.at[pl.ds((NS - 1) * ROWS_A, ROWS_B)],
                out_ref.at[pl.ds((NS - 1) * ROWS_A, ROWS_B),
                           pl.ds(c * HALF, HALF)])

    return sc_spmm


def kernel(x, edge_index, edge_weight, W, b):
    support = _tc_support(x, W).reshape(NC * N_NODES, HALF)

    e = edge_index.shape[1]
    cpt = -(-e // (NS * CHUNK))
    e_pad = NS * cpt * CHUNK
    ei = jnp.pad(edge_index, ((0, 0), (0, e_pad - e)))
    ew = jnp.pad(edge_weight, (0, e_pad - e)).reshape(NS * cpt, CHUNK)
    src = ei[0].reshape(NS * cpt, CHUNK)
    srcs = jnp.stack([src, src + N_NODES])
    dst = ei[1].reshape(NS * cpt, CHUNK)

    return _make_sc_spmm(cpt)(srcs, dst, ew, b, support)


# speedup vs baseline: 5.7144x; 1.0415x over previous
import functools

import jax
import jax.numpy as jnp
from jax import lax
from jax.experimental import pallas as pl
from jax.experimental.pallas import tpu as pltpu
from jax.experimental.pallas import tpu_sc as plsc

N_NODES = 10000
D_IN = 256
D_OUT = 256
HALF = 128
NC, NS = 2, 16
CHUNK = 128
RING = 4
LANES = 16
ROWS_A = 624
ROWS_B = 640

_BCAST_DNUMS = lax.GatherDimensionNumbers(
    offset_dims=(), collapsed_slice_dims=(0,), start_index_map=(0,))


def _lane_broadcast(vec, lane):
    idx = jnp.full((LANES, 1), lane, jnp.int32)
    return lax.gather(vec, idx, _BCAST_DNUMS, (1,),
                      mode=lax.GatherScatterMode.PROMISE_IN_BOUNDS)


def _matmul_body(x_ref, w_ref, out_ref):
    res = jnp.dot(x_ref[...], w_ref[...],
                  preferred_element_type=jnp.float32)
    out_ref[0] = res[:, :HALF]
    out_ref[1] = res[:, HALF:]


def _tc_support(x, W):
    br = 1000
    return pl.pallas_call(
        _matmul_body,
        grid=(N_NODES // br,),
        in_specs=[
            pl.BlockSpec((br, D_IN), lambda i: (i, 0)),
            pl.BlockSpec((D_IN, D_OUT), lambda i: (0, 0)),
        ],
        out_specs=pl.BlockSpec((NC, br, HALF), lambda i: (0, i, 0)),
        out_shape=jax.ShapeDtypeStruct((NC, N_NODES, HALF), jnp.float32),
    )(x, W)


def _make_sc_spmm(cpt):
    mesh = plsc.VectorSubcoreMesh(core_axis_name="c", subcore_axis_name="s",
                                  num_cores=NC, num_subcores=NS)

    @functools.partial(
        pl.kernel,
        out_type=jax.ShapeDtypeStruct((N_NODES, D_OUT), jnp.float32),
        mesh=mesh,
        scratch_types=[
            pltpu.VMEM_SHARED((N_NODES, HALF), jnp.float32),
            pltpu.VMEM((CHUNK, HALF), jnp.float32),
            pltpu.VMEM((CHUNK, HALF), jnp.float32),
            pltpu.VMEM((CHUNK, HALF), jnp.float32),
            pltpu.VMEM((RING, CHUNK), jnp.int32),
            pltpu.VMEM((RING, CHUNK), jnp.int32),
            pltpu.VMEM((RING, CHUNK), jnp.float32),
            pltpu.VMEM((HALF,), jnp.float32),
            pltpu.SemaphoreType.DMA,
            pltpu.SemaphoreType.DMA,
            pltpu.SemaphoreType.DMA,
            pltpu.SemaphoreType.DMA,
            pltpu.SemaphoreType.DMA,
        ],
    )
    def sc_spmm(src_ref, dst_ref, ew_ref, b_ref, sup0_ref, sup1_ref,
                out_ref, acc, b0, b1, b2, src_g, dst_g, w_g, bbuf,
                gsem, ssem, isem_s, isem_d, isem_w):
        c = lax.axis_index("c")
        s = lax.axis_index("s")
        bufs = (b0, b1, b2)
        row0 = s * ROWS_A

        def fire_idx(k):
            slot = k & 3
            base = s * cpt + k
            pltpu.async_copy(src_ref.at[base], src_g.at[slot], isem_s)
            pltpu.async_copy(dst_ref.at[base], dst_g.at[slot], isem_d)
            pltpu.async_copy(ew_ref.at[base], w_g.at[slot], isem_w)

        def wait_idx():
            pltpu.make_async_copy(src_ref.at[0], src_g.at[0],
                                  isem_s).wait()
            pltpu.make_async_copy(dst_ref.at[0], dst_g.at[0], isem_d).wait()
            pltpu.make_async_copy(ew_ref.at[0], w_g.at[0], isem_w).wait()

        def fire_gather(slot, buf):
            @pl.when(c == 0)
            def _():
                pltpu.async_copy(sup0_ref.at[src_g.at[slot]], buf, gsem)

            @pl.when(c == 1)
            def _():
                pltpu.async_copy(sup1_ref.at[src_g.at[slot]], buf, gsem)

        def wait_gather(buf):
            pltpu.make_async_copy(sup0_ref.at[src_g.at[0]], buf, gsem).wait()

        def wait_scatter():
            pltpu.make_async_copy(b0, acc.at[dst_g.at[0]], ssem).wait()

        pltpu.sync_copy(b_ref.at[pl.ds(c * HALF, HALF)], bbuf)
        bv = [bbuf[pl.ds(c8 * LANES, LANES)] for c8 in range(HALF // LANES)]

        @pl.loop(0, CHUNK)
        def _fill(r):
            for c8 in range(HALF // LANES):
                b0[r, pl.ds(c8 * LANES, LANES)] = bv[c8]

        @pl.when(s < NS - 1)
        def _():
            for k in range(ROWS_A // CHUNK):
                pltpu.sync_copy(b0, acc.at[pl.ds(row0 + k * CHUNK, CHUNK)])
            rem = ROWS_A % CHUNK
            pltpu.sync_copy(
                b0.at[pl.ds(0, rem)],
                acc.at[pl.ds(row0 + (ROWS_A // CHUNK) * CHUNK, rem)])

        @pl.when(s == NS - 1)
        def _():
            for k in range(ROWS_B // CHUNK):
                pltpu.sync_copy(
                    b0, acc.at[pl.ds((NS - 1) * ROWS_A + k * CHUNK, CHUNK)])

        plsc.subcore_barrier()

        fire_idx(0)
        fire_idx(1)
        fire_idx(2)
        wait_idx()
        fire_gather(0, b0)
        wait_idx()
        fire_gather(1, b1)

        @pl.loop(0, cpt)
        def _step(k):
            for i in range(3):
                @pl.when(lax.rem(k, 3) == i)
                def _(i=i):
                    buf = bufs[i]
                    nbuf = bufs[(i + 2) % 3]

                    wait_gather(buf)
                    slot = k & 3

                    @pl.loop(0, CHUNK // LANES)
                    def _scale(g):
                        wgrp = w_g[slot, pl.ds(g * LANES, LANES)]
                        for u in range(LANES):
                            e = g * LANES + u
                            wv = _lane_broadcast(wgrp, u)
                            for c8 in range(HALF // LANES):
                                sl = pl.ds(c8 * LANES, LANES)
                                buf[e, sl] = buf[e, sl] * wv

                    @pl.when(k >= 1)
                    def _():
                        wait_scatter()

                    @pl.when(k + 3 < cpt)
                    def _():
                        fire_idx(k + 3)

                    @pl.when(k + 2 < cpt)
                    def _():
                        wait_idx()
                        fire_gather((k + 2) & 3, nbuf)

                    pltpu.async_copy(buf, acc.at[dst_g.at[slot]], ssem,
                                     add=True)

        wait_scatter()
        plsc.subcore_barrier()

        @pl.when(s < NS - 1)
        def _():
            pltpu.sync_copy(
                acc.at[pl.ds(row0, ROWS_A)],
                out_ref.at[pl.ds(row0, ROWS_A), pl.ds(c * HALF, HALF)])

        @pl.when(s == NS - 1)
        def _():
            pltpu.sync_copy(
                acc.at[pl.ds((NS - 1) * ROWS_A, ROWS_B)],
                out_ref.at[pl.ds((NS - 1) * ROWS_A, ROWS_B),
                           pl.ds(c * HALF, HALF)])

    return sc_spmm


def kernel(x, edge_index, edge_weight, W, b):
    support = _tc_support(x, W)

    e = edge_index.shape[1]
    cpt = -(-e // (NS * CHUNK))
    e_pad = NS * cpt * CHUNK
    ei = jnp.pad(edge_index, ((0, 0), (0, e_pad - e)))
    ew = jnp.pad(edge_weight, (0, e_pad - e)).reshape(NS * cpt, CHUNK)
    src = ei[0].reshape(NS * cpt, CHUNK)
    dst = ei[1].reshape(NS * cpt, CHUNK)

    return _make_sc_spmm(cpt)(src, dst, ew, b, support[0], support[1])
